# baseline (speedup 1.0000x reference)
import functools

import numpy as np
import jax
import jax.numpy as jnp
from jax import lax
from jax.experimental import pallas as pl
from jax.experimental.pallas import tpu as pltpu
from jax.experimental.pallas import tpu_sc as plsc

_MASSES = np.array([0.0,1.008,4.0026,6.94,9.0122,10.81,12.011,14.007,15.999,18.998,20.18,22.99,24.305,26.982,28.085,30.974,32.06,35.45,39.948,39.098,40.078,44.956,47.867,50.942,51.996,54.938,55.845,58.933,58.693,63.546,65.38,69.723,72.63,74.922,78.971,79.904,83.798,85.468,87.62,88.906,91.224,92.906,95.95,97.907,101.07,102.906,106.42,107.868,112.414,114.818,118.71,121.76,127.6,126.904,131.293,132.905,137.327,138.905,140.116,140.908,144.242,144.913,150.36,151.964,157.25,158.925,162.5,164.93,167.259,168.934,173.054,174.967,178.49,180.948,183.84,186.207,190.23,192.217,195.084,196.967,200.592,204.38,207.2,208.98,208.982,209.987,222.018,223.02,226.025,227.028,232.038,231.036,238.029,237.048,244.064,243.061,247.07,247.07,251.08,252.083], dtype=np.float32)

_B = 16
_NZ = 100
_R = 4096
_L = 16
_NC = 1
_NW = 16 * _NC


def _silu(x):
    t = 0.5 * x
    return t * jnp.tanh(t) + t


def _softplus(x):
    return jnp.maximum(x, 0.0) + jnp.log(1.0 + jnp.exp(-jnp.abs(x)))


def _tc_block(xa_ref, xb_ref, post_ref, z_ref, bseg_ref, w1_ref, b1_ref,
              w2_ref, b2_ref, wo_ref, tab_ref, out_ref):
    def _q0_half(x):
        h = jnp.dot(x, w1_ref[...],
                    preferred_element_type=jnp.float32) + b1_ref[...]
        h = _silu(h)
        h = jnp.dot(h, w2_ref[...],
                    preferred_element_type=jnp.float32) + b2_ref[...]
        h = _silu(h)
        return lax.dot_general(wo_ref[...], x + h, (((1,), (1,)), ((), ())),
                               preferred_element_type=jnp.float32)

    q0t = jnp.concatenate([_q0_half(xa_ref[...]), _q0_half(xb_ref[...])],
                          axis=1)

    post = post_ref[...]
    rows = post.shape[1]
    zt = z_ref[...]
    bt = bseg_ref[...]

    zoh = (zt == lax.broadcasted_iota(jnp.int32, (_NZ, rows), 0)
           ).astype(jnp.float32)
    refz = jnp.dot(tab_ref[...], zoh, preferred_element_type=jnp.float32)
    qt = _softplus(q0t + refz)

    px = post[0:1, :]
    py = post[1:2, :]
    pz = post[2:3, :]
    r2 = px * px + py * py + pz * pz
    ut = jnp.concatenate([post, r2, jnp.ones_like(r2)], axis=0)

    soh = (bt == lax.broadcasted_iota(jnp.int32, (_B, rows), 0)
           ).astype(jnp.float32)
    stats = jnp.concatenate([qt * ut, jnp.ones_like(r2)], axis=0)
    part = lax.dot_general(soh, stats, (((1,), (1,)), ((), ())),
                           preferred_element_type=jnp.float32)

    @pl.when(pl.program_id(0) == 0)
    def _init():
        out_ref[...] = jnp.zeros_like(out_ref)

    out_ref[...] += part


def _sc_stats_body(post_hbm, z_hbm, b_hbm, tab_hbm, out_hbm,
                   px_v, py_v, pz_v, z_v, b_v, tab_v, acc_v, tot_v, sem):
    chunk = px_v.shape[0]
    wid = lax.axis_index("s") * _NC + lax.axis_index("c")
    base = wid * chunk
    copies = [
        pltpu.async_copy(post_hbm.at[0, 0, pl.ds(base, chunk)], px_v, sem),
        pltpu.async_copy(post_hbm.at[1, 0, pl.ds(base, chunk)], py_v, sem),
        pltpu.async_copy(post_hbm.at[2, 0, pl.ds(base, chunk)], pz_v, sem),
        pltpu.async_copy(z_hbm.at[0, pl.ds(base, chunk)], z_v, sem),
        pltpu.async_copy(b_hbm.at[0, pl.ds(base, chunk)], b_v, sem),
        pltpu.async_copy(tab_hbm, tab_v, sem),
    ]

    zeros = jnp.zeros((_L,), jnp.float32)
    for k in range(8):
        for j in range(_L):
            acc_v[k, pl.ds(j * _L, _L)] = zeros

    for c in copies:
        c.wait()

    lane16 = lax.iota(jnp.int32, _L) * _L

    def body(i, carry):
        for s in range(2):
            off = (2 * i + s) * _L
            zv = z_v[pl.ds(off, _L)]
            bv = b_v[pl.ds(off, _L)]
            pxv = px_v[pl.ds(off, _L)]
            pyv = py_v[pl.ds(off, _L)]
            pzv = pz_v[pl.ds(off, _L)]
            m = plsc.load_gather(tab_v, [zv])
            vidx = lane16 + bv
            for k, val in ((0, m), (1, m * pxv), (2, m * pyv),
                           (3, m * pzv)):
                plsc.addupdate_scatter(
                    acc_v, [jnp.full((_L,), 4 * s + k, jnp.int32), vidx],
                    val)
        return carry

    lax.fori_loop(0, chunk // (2 * _L), body, 0)

    for k in range(4):
        tot = acc_v[k, pl.ds(0, _L)] + acc_v[k + 4, pl.ds(0, _L)]
        for j in range(1, _L):
            tot = tot + (acc_v[k, pl.ds(j * _L, _L)]
                         + acc_v[k + 4, pl.ds(j * _L, _L)])
        tot_v[k, :] = tot
    pltpu.sync_copy(tot_v, out_hbm.at[wid])


def _sc_stats(post, zr, br, tab):
    n = post.shape[1]
    chunk = n // _NW
    post = post.reshape(3, 1, n)
    mesh = plsc.VectorSubcoreMesh(core_axis_name="c", subcore_axis_name="s",
                                  num_cores=_NC, num_subcores=16)
    return pl.kernel(
        _sc_stats_body,
        out_type=jax.ShapeDtypeStruct((_NW, 4, _L), jnp.float32),
        mesh=mesh,
        compiler_params=pltpu.CompilerParams(needs_layout_passes=False),
        scratch_types=[
            pltpu.VMEM((chunk,), jnp.float32),
            pltpu.VMEM((chunk,), jnp.float32),
            pltpu.VMEM((chunk,), jnp.float32),
            pltpu.VMEM((chunk,), jnp.int32),
            pltpu.VMEM((chunk,), jnp.int32),
            pltpu.VMEM((128,), jnp.float32),
            pltpu.VMEM((8, _L * _L), jnp.float32),
            pltpu.VMEM((4, _L), jnp.float32),
            pltpu.SemaphoreType.DMA,
        ],
    )(post, zr, br, tab)


def kernel(kemb, pos, z, batch_index, W1, b1, W2, b2, W_out, ref_table):
    n, h = kemb.shape
    post = pos.T
    zr = z.astype(jnp.int32).reshape(1, n)
    br = batch_index.astype(jnp.int32).reshape(1, n)
    ref0 = ref_table.at[0].set(0.0)
    mass_tab = jnp.pad(jnp.asarray(_MASSES), (0, 28))

    sc_part = _sc_stats(post, zr, br, mass_tab)

    tsums = pl.pallas_call(
        _tc_block,
        grid=(n // _R,),
        in_specs=[
            pl.BlockSpec((_R // 2, h), lambda i: (2 * i, 0)),
            pl.BlockSpec((_R // 2, h), lambda i: (2 * i + 1, 0)),
            pl.BlockSpec((3, _R), lambda i: (0, i)),
            pl.BlockSpec((1, _R), lambda i: (0, i)),
            pl.BlockSpec((1, _R), lambda i: (0, i)),
            pl.BlockSpec((h, h), lambda i: (0, 0)),
            pl.BlockSpec((1, h), lambda i: (0, 0)),
            pl.BlockSpec((h, h), lambda i: (0, 0)),
            pl.BlockSpec((1, h), lambda i: (0, 0)),
            pl.BlockSpec((1, h), lambda i: (0, 0)),
            pl.BlockSpec((1, _NZ), lambda i: (0, 0)),
        ],
        out_specs=pl.BlockSpec((_B, 6), lambda i: (0, 0)),
        out_shape=jax.ShapeDtypeStruct((_B, 6), jnp.float32),
        compiler_params=pltpu.CompilerParams(
            dimension_semantics=("arbitrary",)),
    )(kemb, kemb, post, zr, br, W1, b1[None, :], W2, b2[None, :],
      W_out.T, ref0.T)

    sc = jnp.sum(sc_part, axis=0)
    s0 = sc[0]
    s1 = sc[1:4]
    t1 = tsums[:, 0:3]
    t2 = tsums[:, 3]
    t0 = tsums[:, 4]
    cnt = tsums[:, 5]
    com = s1 / s0
    res = (t2 - 2.0 * jnp.sum(com.T * t1, axis=1)
           + jnp.sum(com * com, axis=0) * t0)
    return jnp.where(cnt > 0, res, 0.0)


# speedup vs baseline: 1.0308x; 1.0308x over previous
import functools

import numpy as np
import jax
import jax.numpy as jnp
from jax import lax
from jax.experimental import pallas as pl
from jax.experimental.pallas import tpu as pltpu
from jax.experimental.pallas import tpu_sc as plsc

_MASSES = np.array([0.0,1.008,4.0026,6.94,9.0122,10.81,12.011,14.007,15.999,18.998,20.18,22.99,24.305,26.982,28.085,30.974,32.06,35.45,39.948,39.098,40.078,44.956,47.867,50.942,51.996,54.938,55.845,58.933,58.693,63.546,65.38,69.723,72.63,74.922,78.971,79.904,83.798,85.468,87.62,88.906,91.224,92.906,95.95,97.907,101.07,102.906,106.42,107.868,112.414,114.818,118.71,121.76,127.6,126.904,131.293,132.905,137.327,138.905,140.116,140.908,144.242,144.913,150.36,151.964,157.25,158.925,162.5,164.93,167.259,168.934,173.054,174.967,178.49,180.948,183.84,186.207,190.23,192.217,195.084,196.967,200.592,204.38,207.2,208.98,208.982,209.987,222.018,223.02,226.025,227.028,232.038,231.036,238.029,237.048,244.064,243.061,247.07,247.07,251.08,252.083], dtype=np.float32)

_B = 16
_NZ = 100
_R = 2048
_L = 16
_NC = 1
_NW = 16 * _NC


def _silu(x):
    t = 0.5 * x
    return t * jnp.tanh(t) + t


def _softplus(x):
    return jnp.maximum(x, 0.0) + jnp.log(1.0 + jnp.exp(-jnp.abs(x)))


def _tc_block(xa_ref, xb_ref, post_ref, z_ref, bseg_ref, w1_ref, b1_ref,
              w2_ref, b2_ref, wo_ref, tab_ref, out_ref):
    def _q0_half(x):
        h = jnp.dot(x, w1_ref[...],
                    preferred_element_type=jnp.float32) + b1_ref[...]
        h = _silu(h)
        h = jnp.dot(h, w2_ref[...],
                    preferred_element_type=jnp.float32) + b2_ref[...]
        h = _silu(h)
        return lax.dot_general(wo_ref[...], x + h, (((1,), (1,)), ((), ())),
                               preferred_element_type=jnp.float32)

    q0t = jnp.concatenate([_q0_half(xa_ref[...]), _q0_half(xb_ref[...])],
                          axis=1)

    post = post_ref[...]
    rows = post.shape[1]
    zt = z_ref[...]
    bt = bseg_ref[...]

    zoh = (zt == lax.broadcasted_iota(jnp.int32, (_NZ, rows), 0)
           ).astype(jnp.float32)
    refz = jnp.dot(tab_ref[...], zoh, preferred_element_type=jnp.float32)
    qt = _softplus(q0t + refz)

    px = post[0:1, :]
    py = post[1:2, :]
    pz = post[2:3, :]
    r2 = px * px + py * py + pz * pz
    ut = jnp.concatenate([post, r2, jnp.ones_like(r2)], axis=0)

    soh = (bt == lax.broadcasted_iota(jnp.int32, (_B, rows), 0)
           ).astype(jnp.float32)
    stats = jnp.concatenate([qt * ut, jnp.ones_like(r2)], axis=0)
    part = lax.dot_general(soh, stats, (((1,), (1,)), ((), ())),
                           preferred_element_type=jnp.float32)

    @pl.when(pl.program_id(0) == 0)
    def _init():
        out_ref[...] = jnp.zeros_like(out_ref)

    out_ref[...] += part


def _sc_stats_body(post_hbm, z_hbm, b_hbm, tab_hbm, out_hbm,
                   px_v, py_v, pz_v, z_v, b_v, tab_v, acc_v, tot_v, sem):
    chunk = px_v.shape[0]
    wid = lax.axis_index("s") * _NC + lax.axis_index("c")
    base = wid * chunk
    copies = [
        pltpu.async_copy(post_hbm.at[0, 0, pl.ds(base, chunk)], px_v, sem),
        pltpu.async_copy(post_hbm.at[1, 0, pl.ds(base, chunk)], py_v, sem),
        pltpu.async_copy(post_hbm.at[2, 0, pl.ds(base, chunk)], pz_v, sem),
        pltpu.async_copy(z_hbm.at[0, pl.ds(base, chunk)], z_v, sem),
        pltpu.async_copy(b_hbm.at[0, pl.ds(base, chunk)], b_v, sem),
        pltpu.async_copy(tab_hbm, tab_v, sem),
    ]

    zeros = jnp.zeros((_L,), jnp.float32)
    for k in range(8):
        for j in range(_L):
            acc_v[k, pl.ds(j * _L, _L)] = zeros

    for c in copies:
        c.wait()

    lane16 = lax.iota(jnp.int32, _L) * _L

    def body(i, carry):
        for s in range(2):
            off = (2 * i + s) * _L
            zv = z_v[pl.ds(off, _L)]
            bv = b_v[pl.ds(off, _L)]
            pxv = px_v[pl.ds(off, _L)]
            pyv = py_v[pl.ds(off, _L)]
            pzv = pz_v[pl.ds(off, _L)]
            m = plsc.load_gather(tab_v, [zv])
            vidx = lane16 + bv
            for k, val in ((0, m), (1, m * pxv), (2, m * pyv),
                           (3, m * pzv)):
                plsc.addupdate_scatter(
                    acc_v, [jnp.full((_L,), 4 * s + k, jnp.int32), vidx],
                    val)
        return carry

    lax.fori_loop(0, chunk // (2 * _L), body, 0)

    for k in range(4):
        tot = acc_v[k, pl.ds(0, _L)] + acc_v[k + 4, pl.ds(0, _L)]
        for j in range(1, _L):
            tot = tot + (acc_v[k, pl.ds(j * _L, _L)]
                         + acc_v[k + 4, pl.ds(j * _L, _L)])
        tot_v[k, :] = tot
    pltpu.sync_copy(tot_v, out_hbm.at[wid])


def _sc_stats(post, zr, br, tab):
    n = post.shape[1]
    chunk = n // _NW
    post = post.reshape(3, 1, n)
    mesh = plsc.VectorSubcoreMesh(core_axis_name="c", subcore_axis_name="s",
                                  num_cores=_NC, num_subcores=16)
    return pl.kernel(
        _sc_stats_body,
        out_type=jax.ShapeDtypeStruct((_NW, 4, _L), jnp.float32),
        mesh=mesh,
        compiler_params=pltpu.CompilerParams(needs_layout_passes=False),
        scratch_types=[
            pltpu.VMEM((chunk,), jnp.float32),
            pltpu.VMEM((chunk,), jnp.float32),
            pltpu.VMEM((chunk,), jnp.float32),
            pltpu.VMEM((chunk,), jnp.int32),
            pltpu.VMEM((chunk,), jnp.int32),
            pltpu.VMEM((128,), jnp.float32),
            pltpu.VMEM((8, _L * _L), jnp.float32),
            pltpu.VMEM((4, _L), jnp.float32),
            pltpu.SemaphoreType.DMA,
        ],
    )(post, zr, br, tab)


def kernel(kemb, pos, z, batch_index, W1, b1, W2, b2, W_out, ref_table):
    n, h = kemb.shape
    post = pos.T
    zr = z.astype(jnp.int32).reshape(1, n)
    br = batch_index.astype(jnp.int32).reshape(1, n)
    ref0 = ref_table.at[0].set(0.0)
    mass_tab = jnp.pad(jnp.asarray(_MASSES), (0, 28))

    sc_part = _sc_stats(post, zr, br, mass_tab)

    tsums = pl.pallas_call(
        _tc_block,
        grid=(n // _R,),
        in_specs=[
            pl.BlockSpec((_R // 2, h), lambda i: (2 * i, 0)),
            pl.BlockSpec((_R // 2, h), lambda i: (2 * i + 1, 0)),
            pl.BlockSpec((3, _R), lambda i: (0, i)),
            pl.BlockSpec((1, _R), lambda i: (0, i)),
            pl.BlockSpec((1, _R), lambda i: (0, i)),
            pl.BlockSpec((h, h), lambda i: (0, 0)),
            pl.BlockSpec((1, h), lambda i: (0, 0)),
            pl.BlockSpec((h, h), lambda i: (0, 0)),
            pl.BlockSpec((1, h), lambda i: (0, 0)),
            pl.BlockSpec((1, h), lambda i: (0, 0)),
            pl.BlockSpec((1, _NZ), lambda i: (0, 0)),
        ],
        out_specs=pl.BlockSpec((_B, 6), lambda i: (0, 0)),
        out_shape=jax.ShapeDtypeStruct((_B, 6), jnp.float32),
        compiler_params=pltpu.CompilerParams(
            dimension_semantics=("arbitrary",)),
    )(kemb, kemb, post, zr, br, W1, b1[None, :], W2, b2[None, :],
      W_out.T, ref0.T)

    sc = jnp.sum(sc_part, axis=0)
    s0 = sc[0]
    s1 = sc[1:4]
    t1 = tsums[:, 0:3]
    t2 = tsums[:, 3]
    t0 = tsums[:, 4]
    cnt = tsums[:, 5]
    com = s1 / s0
    res = (t2 - 2.0 * jnp.sum(com.T * t1, axis=1)
           + jnp.sum(com * com, axis=0) * t0)
    return jnp.where(cnt > 0, res, 0.0)
